# baseline (speedup 1.0000x reference)
import functools

import jax
import jax.numpy as jnp
from jax import lax
from jax.experimental import pallas as pl
from jax.experimental.pallas import tpu as pltpu
from jax.experimental.pallas import tpu_sc as plsc

N = 10000
D = 128
E = 320000
NC = 2
NS = 16
NW = NC * NS
K = 64
C = 162
EPW = C * K
E_PAD = NW * EPW
ZR = 632
ZL = N - 15 * ZR

_F32 = jnp.float32
_I32 = jnp.int32


def _sc_propagate(agg, row_r, col_r, w_r, zeros):
    mesh = plsc.VectorSubcoreMesh(core_axis_name="c", subcore_axis_name="s")

    @functools.partial(
        pl.kernel,
        out_type=jax.ShapeDtypeStruct((NC, N, D), _F32),
        mesh=mesh,
        scratch_types=[
            pltpu.VMEM((C // 2, 2 * K), _I32),
            pltpu.VMEM((C // 2, 2 * K), _I32),
            pltpu.VMEM((3, K), _F32),
            pltpu.VMEM((3, K, D), _F32),
            pltpu.VMEM_SHARED((N, D), _F32),
        ] + [pltpu.SemaphoreType.DMA] * 9,
    )
    def run(agg_hbm, row_hbm, col_hbm, w_hbm, zeros_hbm, acc_hbm,
            row_v, col_v, w_v, rows_v, acc_sh, *sems):
        gsem = sems[0:3]
        wsem = sems[3:6]
        ssem = sems[6:9]
        cid = lax.axis_index("c")
        sid = lax.axis_index("s")
        wid = sid * NC + cid

        @pl.when(sid < NS - 1)
        def _():
            pltpu.async_copy(zeros_hbm.at[pl.ds(0, ZR)],
                             acc_sh.at[pl.ds(sid * ZR, ZR)], ssem[0])

        @pl.when(sid == NS - 1)
        def _():
            pltpu.async_copy(zeros_hbm.at[pl.ds(0, ZL)],
                             acc_sh.at[pl.ds(15 * ZR, ZL)], ssem[0])

        pltpu.async_copy(row_hbm.at[wid], row_v, ssem[1])
        pltpu.async_copy(col_hbm.at[wid], col_v, ssem[2])
        pltpu.make_async_copy(row_hbm.at[wid], row_v, ssem[1]).wait()
        pltpu.make_async_copy(col_hbm.at[wid], col_v, ssem[2]).wait()

        @pl.when(sid < NS - 1)
        def _():
            pltpu.make_async_copy(zeros_hbm.at[pl.ds(0, ZR)],
                                  acc_sh.at[pl.ds(sid * ZR, ZR)],
                                  ssem[0]).wait()

        @pl.when(sid == NS - 1)
        def _():
            pltpu.make_async_copy(zeros_hbm.at[pl.ds(0, ZL)],
                                  acc_sh.at[pl.ds(15 * ZR, ZL)],
                                  ssem[0]).wait()
        plsc.subcore_barrier()

        def idx_slice(ref, c):
            return ref.at[lax.div(c, 2), pl.ds(lax.rem(c, 2) * K, K)]

        def w_slice(c):
            return w_hbm.at[wid, lax.div(c, 2), pl.ds(lax.rem(c, 2) * K, K)]

        def gather(c, p):
            pltpu.async_copy(agg_hbm.at[idx_slice(row_v, c)],
                             rows_v.at[p], gsem[p])
            pltpu.async_copy(w_slice(c), w_v.at[p], wsem[p])

        def scatter_wait(p):
            pltpu.make_async_copy(
                rows_v.at[p], acc_sh.at[idx_slice(col_v, 0)], ssem[p]).wait()

        def step(c, p):
            pn = (p + 1) % 3

            @pl.when(c >= 2)
            def _():
                scatter_wait(pn)

            @pl.when(c + 1 < C)
            def _():
                gather(c + 1, pn)

            pltpu.make_async_copy(
                agg_hbm.at[idx_slice(row_v, c)], rows_v.at[p],
                gsem[p]).wait()
            pltpu.make_async_copy(w_slice(c), w_v.at[p], wsem[p]).wait()

            for g in range(K // 16):
                wv = w_v[p, pl.ds(g * 16, 16)]
                for j in range(16):
                    w16 = jnp.full((16,), wv[j], dtype=_F32)
                    e = g * 16 + j
                    for f in range(D // 16):
                        x = rows_v[p, e, pl.ds(f * 16, 16)]
                        rows_v[p, e, pl.ds(f * 16, 16)] = x * w16

            pltpu.async_copy(rows_v.at[p], acc_sh.at[idx_slice(col_v, c)],
                             ssem[p], add=True)

        gather(0, 0)

        def chunktriple(t, carry):
            step(t * 3, 0)
            step(t * 3 + 1, 1)
            step(t * 3 + 2, 2)
            return carry

        lax.fori_loop(0, C // 3, chunktriple, 0)
        scatter_wait((C - 2) % 3)
        scatter_wait((C - 1) % 3)
        plsc.subcore_barrier()

        @pl.when(sid < NS - 1)
        def _():
            pltpu.sync_copy(acc_sh.at[pl.ds(sid * ZR, ZR)],
                            acc_hbm.at[cid, pl.ds(sid * ZR, ZR)])

        @pl.when(sid == NS - 1)
        def _():
            pltpu.sync_copy(acc_sh.at[pl.ds(15 * ZR, ZL)],
                            acc_hbm.at[cid, pl.ds(15 * ZR, ZL)])

    return run(agg, row_r, col_r, w_r, zeros)


def _tc_dense(acc2, embed, prev, cum, W_mlp, b_mlp, W1a, W1b, b1, W2b, b2b):
    BN = 2000
    grid = (N // BN,)

    def body(acc_ref, e_ref, p_ref, c_ref, wm_ref, bm_ref,
             w1a_ref, w1b_ref, b1_ref, w2_ref, b2_ref, new_ref, cum_ref):
        s = acc_ref[0] + acc_ref[1]
        e = e_ref[...]
        h = (jnp.dot(s, w1a_ref[...], preferred_element_type=_F32)
             + jnp.dot(e, w1b_ref[...], preferred_element_type=_F32)
             + b1_ref[...])
        h = jnp.maximum(h, 0.0)
        z = jnp.dot(h, w2_ref[...], preferred_element_type=_F32) + b2_ref[...]
        g = 1.0 / (1.0 + jnp.exp(-z))
        cn = c_ref[...] + g
        new = (jnp.dot(s, wm_ref[...], preferred_element_type=_F32)
               + bm_ref[...] + s + (1.0 - cn) * e + p_ref[...])
        new_ref[...] = new
        cum_ref[...] = cn

    full = lambda shape: pl.BlockSpec(shape, lambda i: tuple(0 for _ in shape))
    return pl.pallas_call(
        body,
        grid=grid,
        in_specs=[
            pl.BlockSpec((NC, BN, D), lambda i: (0, i, 0)),
            pl.BlockSpec((BN, D), lambda i: (i, 0)),
            pl.BlockSpec((BN, D), lambda i: (i, 0)),
            pl.BlockSpec((BN, D), lambda i: (i, 0)),
            full((D, D)),
            full((1, D)),
            full((D, 64)),
            full((D, 64)),
            full((1, 64)),
            full((64, D)),
            full((1, D)),
        ],
        out_specs=[
            pl.BlockSpec((BN, D), lambda i: (i, 0)),
            pl.BlockSpec((BN, D), lambda i: (i, 0)),
        ],
        out_shape=[
            jax.ShapeDtypeStruct((N, D), _F32),
            jax.ShapeDtypeStruct((N, D), _F32),
        ],
    )(acc2, embed, prev, cum, W_mlp, b_mlp, W1a, W1b, b1, W2b, b2b)


def kernel(embed, edge_index, adap_weight, W_mlp, b_mlp, W1, b1, W2, b2):
    row = edge_index[0]
    col = edge_index[1]
    pad = E_PAD - E
    pad_idx = (jnp.arange(pad, dtype=_I32) * 13) % N
    row_r = jnp.concatenate([row, pad_idx]).reshape(NW, C // 2, 2 * K)
    col_r = jnp.concatenate([col, pad_idx]).reshape(NW, C // 2, 2 * K)
    w_r = jnp.concatenate(
        [adap_weight, jnp.zeros((pad,), _F32)]).reshape(NW, C // 2, 2 * K)

    zeros = jnp.zeros((ZR, D), _F32)
    W1a = W1[:D]
    W1b = W1[D:]
    b1r = b1.reshape(1, 64)
    b_mlpr = b_mlp.reshape(1, D)
    W2b = jnp.broadcast_to(W2, (64, D))
    b2b = jnp.broadcast_to(b2.reshape(1, 1), (1, D))

    agg = embed
    cum = jnp.zeros((N, D), _F32)
    embs = [embed]
    for _hop in range(3):
        acc2 = _sc_propagate(agg, row_r, col_r, w_r, zeros)
        agg, cum = _tc_dense(acc2, embed, agg, cum,
                             W_mlp, b_mlpr, W1a, W1b, b1r, W2b, b2b)
        embs.append(agg)
    return jnp.stack(embs, axis=1)


# speedup vs baseline: 1.2528x; 1.2528x over previous
import functools

import jax
import jax.numpy as jnp
from jax import lax
from jax.experimental import pallas as pl
from jax.experimental.pallas import tpu as pltpu
from jax.experimental.pallas import tpu_sc as plsc

N = 10000
D = 128
E = 320000
NC = 2
NS = 16
NW = NC * NS
K = 64
C = 162
EPW = C * K
E_PAD = NW * EPW
ZR = 632
ZL = N - 15 * ZR

_F32 = jnp.float32
_I32 = jnp.int32


def _sc_propagate(agg, row_r, col_r, w_r, zeros):
    mesh = plsc.VectorSubcoreMesh(core_axis_name="c", subcore_axis_name="s")

    @functools.partial(
        pl.kernel,
        out_type=jax.ShapeDtypeStruct((NC, N, D), _F32),
        mesh=mesh,
        scratch_types=[
            pltpu.VMEM((C // 2, 2 * K), _I32),
            pltpu.VMEM((C // 2, 2 * K), _I32),
            pltpu.VMEM((3, K), _F32),
            pltpu.VMEM((3, K, D), _F32),
            pltpu.VMEM_SHARED((N, D), _F32),
        ] + [pltpu.SemaphoreType.DMA] * 9,
    )
    def run(agg_hbm, row_hbm, col_hbm, w_hbm, zeros_hbm, acc_hbm,
            row_v, col_v, w_v, rows_v, acc_sh, *sems):
        gsem = sems[0:3]
        wsem = sems[3:6]
        ssem = sems[6:9]
        cid = lax.axis_index("c")
        sid = lax.axis_index("s")
        wid = sid * NC + cid

        @pl.when(sid < NS - 1)
        def _():
            pltpu.async_copy(zeros_hbm.at[pl.ds(0, ZR)],
                             acc_sh.at[pl.ds(sid * ZR, ZR)], ssem[0])

        @pl.when(sid == NS - 1)
        def _():
            pltpu.async_copy(zeros_hbm.at[pl.ds(0, ZL)],
                             acc_sh.at[pl.ds(15 * ZR, ZL)], ssem[0])

        pltpu.async_copy(row_hbm.at[wid], row_v, ssem[1])
        pltpu.async_copy(col_hbm.at[wid], col_v, ssem[2])
        pltpu.make_async_copy(row_hbm.at[wid], row_v, ssem[1]).wait()
        pltpu.make_async_copy(col_hbm.at[wid], col_v, ssem[2]).wait()

        @pl.when(sid < NS - 1)
        def _():
            pltpu.make_async_copy(zeros_hbm.at[pl.ds(0, ZR)],
                                  acc_sh.at[pl.ds(sid * ZR, ZR)],
                                  ssem[0]).wait()

        @pl.when(sid == NS - 1)
        def _():
            pltpu.make_async_copy(zeros_hbm.at[pl.ds(0, ZL)],
                                  acc_sh.at[pl.ds(15 * ZR, ZL)],
                                  ssem[0]).wait()
        plsc.subcore_barrier()

        def idx_slice(ref, c):
            return ref.at[lax.div(c, 2), pl.ds(lax.rem(c, 2) * K, K)]

        def w_slice(c):
            return w_hbm.at[wid, lax.div(c, 2), pl.ds(lax.rem(c, 2) * K, K)]

        def gather(c, p):
            pltpu.async_copy(agg_hbm.at[idx_slice(row_v, c)],
                             rows_v.at[p], gsem[p])
            pltpu.async_copy(w_slice(c), w_v.at[p], wsem[p])

        def scatter_wait(p):
            pltpu.make_async_copy(
                rows_v.at[p], acc_sh.at[idx_slice(col_v, 0)], ssem[p]).wait()

        def step(c, p):
            pn = (p + 1) % 3

            @pl.when(c >= 2)
            def _():
                scatter_wait(pn)

            @pl.when(c + 1 < C)
            def _():
                gather(c + 1, pn)

            pltpu.make_async_copy(
                agg_hbm.at[idx_slice(row_v, c)], rows_v.at[p],
                gsem[p]).wait()
            pltpu.make_async_copy(w_slice(c), w_v.at[p], wsem[p]).wait()

            @plsc.parallel_loop(0, K // 16, unroll=1)
            def grp(g):
                wv = w_v[p, pl.ds(g * 16, 16)]
                for j in range(16):
                    w16 = jnp.full((16,), wv[j], dtype=_F32)
                    e = g * 16 + j
                    for f in range(D // 16):
                        x = rows_v[p, e, pl.ds(f * 16, 16)]
                        rows_v[p, e, pl.ds(f * 16, 16)] = x * w16

            pltpu.async_copy(rows_v.at[p], acc_sh.at[idx_slice(col_v, c)],
                             ssem[p], add=True)

        gather(0, 0)

        def chunktriple(t, carry):
            step(t * 3, 0)
            step(t * 3 + 1, 1)
            step(t * 3 + 2, 2)
            return carry

        lax.fori_loop(0, C // 3, chunktriple, 0)
        scatter_wait((C - 2) % 3)
        scatter_wait((C - 1) % 3)
        plsc.subcore_barrier()

        @pl.when(sid < NS - 1)
        def _():
            pltpu.sync_copy(acc_sh.at[pl.ds(sid * ZR, ZR)],
                            acc_hbm.at[cid, pl.ds(sid * ZR, ZR)])

        @pl.when(sid == NS - 1)
        def _():
            pltpu.sync_copy(acc_sh.at[pl.ds(15 * ZR, ZL)],
                            acc_hbm.at[cid, pl.ds(15 * ZR, ZL)])

    return run(agg, row_r, col_r, w_r, zeros)


def _tc_dense(acc2, embed, prev, cum, W_mlp, b_mlp, W1a, W1b, b1, W2b, b2b):
    BN = 2000
    grid = (N // BN,)

    def body(acc_ref, e_ref, p_ref, c_ref, wm_ref, bm_ref,
             w1a_ref, w1b_ref, b1_ref, w2_ref, b2_ref, new_ref, cum_ref):
        s = acc_ref[0] + acc_ref[1]
        e = e_ref[...]
        h = (jnp.dot(s, w1a_ref[...], preferred_element_type=_F32)
             + jnp.dot(e, w1b_ref[...], preferred_element_type=_F32)
             + b1_ref[...])
        h = jnp.maximum(h, 0.0)
        z = jnp.dot(h, w2_ref[...], preferred_element_type=_F32) + b2_ref[...]
        g = 1.0 / (1.0 + jnp.exp(-z))
        cn = c_ref[...] + g
        new = (jnp.dot(s, wm_ref[...], preferred_element_type=_F32)
               + bm_ref[...] + s + (1.0 - cn) * e + p_ref[...])
        new_ref[...] = new
        cum_ref[...] = cn

    full = lambda shape: pl.BlockSpec(shape, lambda i: tuple(0 for _ in shape))
    return pl.pallas_call(
        body,
        grid=grid,
        in_specs=[
            pl.BlockSpec((NC, BN, D), lambda i: (0, i, 0)),
            pl.BlockSpec((BN, D), lambda i: (i, 0)),
            pl.BlockSpec((BN, D), lambda i: (i, 0)),
            pl.BlockSpec((BN, D), lambda i: (i, 0)),
            full((D, D)),
            full((1, D)),
            full((D, 64)),
            full((D, 64)),
            full((1, 64)),
            full((64, D)),
            full((1, D)),
        ],
        out_specs=[
            pl.BlockSpec((BN, D), lambda i: (i, 0)),
            pl.BlockSpec((BN, D), lambda i: (i, 0)),
        ],
        out_shape=[
            jax.ShapeDtypeStruct((N, D), _F32),
            jax.ShapeDtypeStruct((N, D), _F32),
        ],
    )(acc2, embed, prev, cum, W_mlp, b_mlp, W1a, W1b, b1, W2b, b2b)


def kernel(embed, edge_index, adap_weight, W_mlp, b_mlp, W1, b1, W2, b2):
    row = edge_index[0]
    col = edge_index[1]
    pad = E_PAD - E
    pad_idx = (jnp.arange(pad, dtype=_I32) * 13) % N
    row_r = jnp.concatenate([row, pad_idx]).reshape(NW, C // 2, 2 * K)
    col_r = jnp.concatenate([col, pad_idx]).reshape(NW, C // 2, 2 * K)
    w_r = jnp.concatenate(
        [adap_weight, jnp.zeros((pad,), _F32)]).reshape(NW, C // 2, 2 * K)

    zeros = jnp.zeros((ZR, D), _F32)
    W1a = W1[:D]
    W1b = W1[D:]
    b1r = b1.reshape(1, 64)
    b_mlpr = b_mlp.reshape(1, D)
    W2b = jnp.broadcast_to(W2, (64, D))
    b2b = jnp.broadcast_to(b2.reshape(1, 1), (1, D))

    agg = embed
    cum = jnp.zeros((N, D), _F32)
    embs = [embed]
    for _hop in range(3):
        acc2 = _sc_propagate(agg, row_r, col_r, w_r, zeros)
        agg, cum = _tc_dense(acc2, embed, agg, cum,
                             W_mlp, b_mlpr, W1a, W1b, b1r, W2b, b2b)
        embs.append(agg)
    return jnp.stack(embs, axis=1)


# speedup vs baseline: 1.2652x; 1.0099x over previous
import functools

import jax
import jax.numpy as jnp
from jax import lax
from jax.experimental import pallas as pl
from jax.experimental.pallas import tpu as pltpu
from jax.experimental.pallas import tpu_sc as plsc

N = 10000
D = 128
E = 320000
NC = 2
NS = 16
NW = NC * NS
K = 64
C = 162
EPW = C * K
E_PAD = NW * EPW
ZR = 632
ZL = N - 15 * ZR

_F32 = jnp.float32
_I32 = jnp.int32


def _sc_propagate(agg, row_r, col_r, w_r, zeros):
    mesh = plsc.VectorSubcoreMesh(core_axis_name="c", subcore_axis_name="s")

    @functools.partial(
        pl.kernel,
        out_type=jax.ShapeDtypeStruct((NC, N, D), _F32),
        mesh=mesh,
        scratch_types=[
            pltpu.VMEM((C // 2, 2 * K), _I32),
            pltpu.VMEM((C // 2, 2 * K), _I32),
            pltpu.VMEM((3, K), _F32),
            pltpu.VMEM((3, K, D), _F32),
            pltpu.VMEM_SHARED((N, D), _F32),
        ] + [pltpu.SemaphoreType.DMA] * 9,
    )
    def run(agg_hbm, row_hbm, col_hbm, w_hbm, zeros_hbm, acc_hbm,
            row_v, col_v, w_v, rows_v, acc_sh, *sems):
        gsem = sems[0:3]
        wsem = sems[3:6]
        ssem = sems[6:9]
        cid = lax.axis_index("c")
        sid = lax.axis_index("s")
        wid = sid * NC + cid

        @pl.when(sid < NS - 1)
        def _():
            pltpu.async_copy(zeros_hbm.at[pl.ds(0, ZR)],
                             acc_sh.at[pl.ds(sid * ZR, ZR)], ssem[0])

        @pl.when(sid == NS - 1)
        def _():
            pltpu.async_copy(zeros_hbm.at[pl.ds(0, ZL)],
                             acc_sh.at[pl.ds(15 * ZR, ZL)], ssem[0])

        pltpu.async_copy(row_hbm.at[wid], row_v, ssem[1])
        pltpu.async_copy(col_hbm.at[wid], col_v, ssem[2])
        pltpu.make_async_copy(row_hbm.at[wid], row_v, ssem[1]).wait()
        pltpu.make_async_copy(col_hbm.at[wid], col_v, ssem[2]).wait()

        @pl.when(sid < NS - 1)
        def _():
            pltpu.make_async_copy(zeros_hbm.at[pl.ds(0, ZR)],
                                  acc_sh.at[pl.ds(sid * ZR, ZR)],
                                  ssem[0]).wait()

        @pl.when(sid == NS - 1)
        def _():
            pltpu.make_async_copy(zeros_hbm.at[pl.ds(0, ZL)],
                                  acc_sh.at[pl.ds(15 * ZR, ZL)],
                                  ssem[0]).wait()
        plsc.subcore_barrier()

        def idx_slice(ref, c):
            return ref.at[lax.div(c, 2), pl.ds(lax.rem(c, 2) * K, K)]

        def w_slice(c):
            return w_hbm.at[wid, lax.div(c, 2), pl.ds(lax.rem(c, 2) * K, K)]

        def gather(c, p):
            pltpu.async_copy(agg_hbm.at[idx_slice(row_v, c)],
                             rows_v.at[p], gsem[p])
            pltpu.async_copy(w_slice(c), w_v.at[p], wsem[p])

        def scatter_wait(p):
            pltpu.make_async_copy(
                rows_v.at[p], acc_sh.at[idx_slice(col_v, 0)], ssem[p]).wait()

        def step(c, p):
            pn = (p + 1) % 3

            @pl.when(c >= 2)
            def _():
                scatter_wait(pn)

            @pl.when(c + 1 < C)
            def _():
                gather(c + 1, pn)

            pltpu.make_async_copy(
                agg_hbm.at[idx_slice(row_v, c)], rows_v.at[p],
                gsem[p]).wait()
            pltpu.make_async_copy(w_slice(c), w_v.at[p], wsem[p]).wait()

            @plsc.parallel_loop(0, K // 16, unroll=1)
            def grp(g):
                wv = w_v[p, pl.ds(g * 16, 16)]
                for j in range(16):
                    w16 = jnp.full((16,), wv[j], dtype=_F32)
                    e = g * 16 + j
                    for f in range(D // 16):
                        x = rows_v[p, e, pl.ds(f * 16, 16)]
                        rows_v[p, e, pl.ds(f * 16, 16)] = x * w16

            pltpu.async_copy(rows_v.at[p], acc_sh.at[idx_slice(col_v, c)],
                             ssem[p], add=True)

        gather(0, 0)

        def chunktriple(t, carry):
            step(t * 3, 0)
            step(t * 3 + 1, 1)
            step(t * 3 + 2, 2)
            return carry

        lax.fori_loop(0, C // 3, chunktriple, 0)
        scatter_wait((C - 2) % 3)
        scatter_wait((C - 1) % 3)
        plsc.subcore_barrier()

        @pl.when(sid < NS - 1)
        def _():
            pltpu.sync_copy(acc_sh.at[pl.ds(sid * ZR, ZR)],
                            acc_hbm.at[cid, pl.ds(sid * ZR, ZR)])

        @pl.when(sid == NS - 1)
        def _():
            pltpu.sync_copy(acc_sh.at[pl.ds(15 * ZR, ZL)],
                            acc_hbm.at[cid, pl.ds(15 * ZR, ZL)])

    return run(agg, row_r, col_r, w_r, zeros)


def _tc_dense(acc2, embed, prev, cum, W_mlp, b_mlp, W1a, W1b, b1, W2b, b2b):
    BN = 2000
    grid = (N // BN,)
    hop0 = prev is None

    def body(*refs):
        if hop0:
            (acc_ref, e_ref, wm_ref, bm_ref, w1a_ref, w1b_ref, b1_ref,
             w2_ref, b2_ref, new_ref, cum_ref) = refs
        else:
            (acc_ref, e_ref, p_ref, c_ref, wm_ref, bm_ref, w1a_ref, w1b_ref,
             b1_ref, w2_ref, b2_ref, new_ref, cum_ref) = refs
        s = acc_ref[0] + acc_ref[1]
        e = e_ref[...]
        h = (jnp.dot(s, w1a_ref[...], preferred_element_type=_F32)
             + jnp.dot(e, w1b_ref[...], preferred_element_type=_F32)
             + b1_ref[...])
        h = jnp.maximum(h, 0.0)
        z = jnp.dot(h, w2_ref[...], preferred_element_type=_F32) + b2_ref[...]
        g = 1.0 / (1.0 + jnp.exp(-z))
        cn = g if hop0 else c_ref[...] + g
        new = (jnp.dot(s, wm_ref[...], preferred_element_type=_F32)
               + bm_ref[...] + s + (1.0 - cn) * e
               + (e if hop0 else p_ref[...]))
        new_ref[...] = new
        cum_ref[...] = cn

    full = lambda shape: pl.BlockSpec(shape, lambda i: tuple(0 for _ in shape))
    node = pl.BlockSpec((BN, D), lambda i: (i, 0))
    in_specs = [pl.BlockSpec((NC, BN, D), lambda i: (0, i, 0)), node]
    args = [acc2, embed]
    if not hop0:
        in_specs += [node, node]
        args += [prev, cum]
    in_specs += [full((D, D)), full((1, D)), full((D, 64)), full((D, 64)),
                 full((1, 64)), full((64, D)), full((1, D))]
    args += [W_mlp, b_mlp, W1a, W1b, b1, W2b, b2b]
    return pl.pallas_call(
        body,
        grid=grid,
        in_specs=in_specs,
        out_specs=[node, node],
        out_shape=[
            jax.ShapeDtypeStruct((N, D), _F32),
            jax.ShapeDtypeStruct((N, D), _F32),
        ],
    )(*args)


def kernel(embed, edge_index, adap_weight, W_mlp, b_mlp, W1, b1, W2, b2):
    row = edge_index[0]
    col = edge_index[1]
    pad = E_PAD - E
    pad_idx = (jnp.arange(pad, dtype=_I32) * 13) % N
    row_r = jnp.concatenate([row, pad_idx]).reshape(NW, C // 2, 2 * K)
    col_r = jnp.concatenate([col, pad_idx]).reshape(NW, C // 2, 2 * K)
    w_r = jnp.concatenate(
        [adap_weight, jnp.zeros((pad,), _F32)]).reshape(NW, C // 2, 2 * K)

    zeros = jnp.zeros((ZR, D), _F32)
    W1a = W1[:D]
    W1b = W1[D:]
    b1r = b1.reshape(1, 64)
    b_mlpr = b_mlp.reshape(1, D)
    W2b = jnp.broadcast_to(W2, (64, D))
    b2b = jnp.broadcast_to(b2.reshape(1, 1), (1, D))

    agg = embed
    cum = None
    prev = None
    embs = [embed]
    for _hop in range(3):
        acc2 = _sc_propagate(agg, row_r, col_r, w_r, zeros)
        prev = agg
        agg, cum = _tc_dense(acc2, embed, None if _hop == 0 else prev,
                             cum, W_mlp, b_mlpr, W1a, W1b, b1r, W2b, b2b)
        embs.append(agg)
    return jnp.stack(embs, axis=1)
